# Initial kernel scaffold; baseline (speedup 1.0000x reference)
import jax
import jax.numpy as jnp
from jax.experimental import pallas as pl


def kernel(Av, params_reac, y_in, x_H2, factor, den_Av_ratio_0):
    raise NotImplementedError("write your pallas kernel here")


# speedup vs baseline: 1366.9975x; 1366.9975x over previous
import functools

import jax
import jax.numpy as jnp
from jax import lax
from jax.experimental import pallas as pl
from jax.experimental.pallas import tpu as pltpu
from jax.experimental.pallas import tpu_sc as plsc

IDX_H2 = 10

NC = 2
NS = 16
L = 16
NW = NC * NS

_LOG2_10 = 3.321928094887362
_S1 = 127.0 / (13.0 * _LOG2_10)
_S0 = -10.0 * _LOG2_10 * _S1


def _make_sc_call(B, K):
    chunk = B // NW
    steps = chunk // L
    mesh = plsc.VectorSubcoreMesh(core_axis_name="c", subcore_axis_name="s",
                                  num_cores=NC, num_subcores=NS)

    @functools.partial(
        pl.kernel,
        out_type=jax.ShapeDtypeStruct((B,), jnp.float32),
        mesh=mesh,
        compiler_params=pltpu.CompilerParams(needs_layout_passes=False),
        scratch_types=[
            pltpu.VMEM((chunk,), jnp.float32),
            pltpu.VMEM((chunk,), jnp.float32),
            pltpu.VMEM((chunk,), jnp.float32),
            pltpu.VMEM((K,), jnp.float32),
            pltpu.VMEM((K,), jnp.float32),
            pltpu.VMEM((L,), jnp.float32),
        ],
    )
    def sc_call(av_hbm, yc_hbm, xt_hbm, fac_hbm, cvec_hbm, out_hbm,
                av_v, yc_v, out_v, xt_v, fac_v, c_v):
        wid = lax.axis_index("s") * NC + lax.axis_index("c")
        base = wid * chunk
        pltpu.sync_copy(xt_hbm, xt_v)
        pltpu.sync_copy(fac_hbm, fac_v)
        pltpu.sync_copy(cvec_hbm, c_v)
        pltpu.sync_copy(av_hbm.at[pl.ds(base, chunk)], av_v)
        pltpu.sync_copy(yc_hbm.at[pl.ds(base, chunk)], yc_v)
        c = c_v[...]

        def step(i, carry):
            sl = pl.ds(i * L, L)
            q = (av_v[sl] * c) * yc_v[sl]
            bits = lax.bitcast_convert_type(q, jnp.int32)
            zf = bits.astype(jnp.float32) * (1.0 / (1 << 23)) - 127.0
            idx_f = jnp.clip(zf * _S1 + _S0, 0.0, float(K - 3))
            j = idx_f.astype(jnp.int32)
            xm = plsc.load_gather(xt_v, [j + 1])
            up = q >= xm
            i0 = jnp.where(up, j + 1, j)
            xo = plsc.load_gather(xt_v, [jnp.where(up, j + 2, j)])
            x0 = jnp.where(up, xm, xo)
            x1 = jnp.where(up, xo, xm)
            f0 = plsc.load_gather(fac_v, [i0])
            f1 = plsc.load_gather(fac_v, [i0 + 1])
            t = jnp.clip((q - x0) / (x1 - x0), 0.0, 1.0)
            out_v[sl] = f0 + (f1 - f0) * t
            return carry

        lax.fori_loop(0, steps, step, 0)
        pltpu.sync_copy(out_v, out_hbm.at[pl.ds(base, chunk)])

    return sc_call


def kernel(Av, params_reac, y_in, x_H2, factor, den_Av_ratio_0):
    B = Av.shape[0]
    K = x_H2.shape[0]
    av = Av.reshape(B)
    yc = y_in[:, IDX_H2]
    fac = factor.reshape(K)
    cvec = jnp.full((L,), den_Av_ratio_0, dtype=jnp.float32)
    out = _make_sc_call(B, K)(av, yc, x_H2, fac, cvec)
    return out.reshape(B, 1)


# speedup vs baseline: 2392.8878x; 1.7505x over previous
import functools

import jax
import jax.numpy as jnp
from jax import lax
from jax.experimental import pallas as pl
from jax.experimental.pallas import tpu as pltpu
from jax.experimental.pallas import tpu_sc as plsc

IDX_H2 = 10

NC = 2
NS = 16
L = 16
NW = NC * NS

_LOG2_10 = 3.321928094887362
_S1 = 127.0 / (13.0 * _LOG2_10)
_S0 = -10.0 * _LOG2_10 * _S1


def _make_sc_call(B, K):
    chunk = B // NW
    steps = chunk // L
    mesh = plsc.VectorSubcoreMesh(core_axis_name="c", subcore_axis_name="s",
                                  num_cores=NC, num_subcores=NS)

    @functools.partial(
        pl.kernel,
        out_type=jax.ShapeDtypeStruct((B,), jnp.float32),
        mesh=mesh,
        compiler_params=pltpu.CompilerParams(needs_layout_passes=False),
        scratch_types=[
            pltpu.VMEM((chunk,), jnp.float32),
            pltpu.VMEM((chunk,), jnp.float32),
            pltpu.VMEM((chunk,), jnp.float32),
            pltpu.VMEM((K,), jnp.float32),
            pltpu.VMEM((K,), jnp.float32),
            pltpu.VMEM((L,), jnp.float32),
        ],
    )
    def sc_call(av_hbm, yc_hbm, xt_hbm, fac_hbm, cvec_hbm, out_hbm,
                av_v, yc_v, out_v, xt_v, fac_v, c_v):
        wid = lax.axis_index("s") * NC + lax.axis_index("c")
        base = wid * chunk
        pltpu.sync_copy(xt_hbm, xt_v)
        pltpu.sync_copy(fac_hbm, fac_v)
        pltpu.sync_copy(cvec_hbm, c_v)
        pltpu.sync_copy(av_hbm.at[pl.ds(base, chunk)], av_v)
        pltpu.sync_copy(yc_hbm.at[pl.ds(base, chunk)], yc_v)
        c = c_v[...]

        def step(i):
            sl = pl.ds(i * L, L)
            q = (av_v[sl] * c) * yc_v[sl]
            bits = lax.bitcast_convert_type(q, jnp.int32)
            zf = bits.astype(jnp.float32) * (1.0 / (1 << 23)) - 127.0
            idx_f = jnp.clip(zf * _S1 + _S0, 0.0, float(K - 3))
            j = idx_f.astype(jnp.int32)
            xm = plsc.load_gather(xt_v, [j + 1])
            up = q >= xm
            i0 = jnp.where(up, j + 1, j)
            xo = plsc.load_gather(xt_v, [jnp.where(up, j + 2, j)])
            x0 = jnp.where(up, xm, xo)
            x1 = jnp.where(up, xo, xm)
            f0 = plsc.load_gather(fac_v, [i0])
            f1 = plsc.load_gather(fac_v, [i0 + 1])
            t = jnp.clip((q - x0) / (x1 - x0), 0.0, 1.0)
            out_v[sl] = f0 + (f1 - f0) * t

        plsc.parallel_loop(0, steps, 1, unroll=8)(step)
        pltpu.sync_copy(out_v, out_hbm.at[pl.ds(base, chunk)])

    return sc_call


def kernel(Av, params_reac, y_in, x_H2, factor, den_Av_ratio_0):
    B = Av.shape[0]
    K = x_H2.shape[0]
    av = Av.reshape(B)
    yc = y_in[:, IDX_H2]
    fac = factor.reshape(K)
    cvec = jnp.full((L,), den_Av_ratio_0, dtype=jnp.float32)
    out = _make_sc_call(B, K)(av, yc, x_H2, fac, cvec)
    return out.reshape(B, 1)
